# baseline (speedup 1.0000x reference)
import functools

import jax
import jax.numpy as jnp
from jax import lax
from jax.experimental import pallas as pl
from jax.experimental.pallas import tpu as pltpu
from jax.experimental.pallas import tpu_sc as plsc

N = 10000
E = 320000
L = 16
NC = 2
NS = 16
NW = NC * NS
NP = 10240
STR = NP // NS
E1 = E // NS
E2 = E // NW

_sc_mesh = plsc.VectorSubcoreMesh(
    core_axis_name="c", subcore_axis_name="s", num_cores=NC, num_subcores=NS)


def _rsqrt16(v):
  i = plsc.bitcast(v, jnp.int32)
  y = plsc.bitcast(jnp.int32(0x5F3759DF) - (i >> 1), jnp.float32)
  for _ in range(3):
    y = y * (1.5 - 0.5 * v * y * y)
  return y


def _sc_main_body(src_hbm, dst_hbm, ew_hbm, hf_hbm, b1_hbm,
                  accp_hbm, dinv2_hbm, degp_hbm, dinvp_hbm,
                  dstv, ewv, srcv, deg, sbuf, dbuf, d2buf, hd, acc, b2,
                  sem_a, sem_b, sem_c, sem_d, sem_e):
  cid = lax.axis_index("c")
  sid = lax.axis_index("s")
  wid = cid * NS + sid
  io = lax.iota(jnp.int32, L)
  half_io = io >> 1

  cp_d = pltpu.async_copy(dst_hbm.at[pl.ds(sid * E1, E1)], dstv, sem_a)
  cp_w = pltpu.async_copy(ew_hbm.at[pl.ds(sid * E1, E1)], ewv, sem_b)
  pltpu.sync_copy(b1_hbm, b2)

  def zero_deg(i, _):
    deg[pl.ds(i * L, L)] = jnp.zeros((L,), jnp.float32)
    return 0
  lax.fori_loop(0, NP // L, zero_deg, 0)
  cp_d.wait()
  cp_w.wait()

  def p1(i, _):
    d = dstv[pl.ds(i * L, L)]
    w = ewv[pl.ds(i * L, L)]
    plsc.addupdate_scatter(deg, [d], w)
    return 0
  lax.fori_loop(0, E1 // L, p1, 0)

  cp_s = pltpu.async_copy(src_hbm.at[pl.ds(wid * E2, E2)], srcv, sem_a)
  cp_d2 = pltpu.async_copy(dst_hbm.at[pl.ds(wid * E2, E2)],
                           dstv.at[pl.ds(0, E2)], sem_b)
  cp_w2 = pltpu.async_copy(ew_hbm.at[pl.ds(wid * E2, E2)],
                           ewv.at[pl.ds(0, E2)], sem_c)
  cp_h = pltpu.async_copy(hf_hbm, hd, sem_d)

  pltpu.sync_copy(deg, degp_hbm.at[pl.ds(wid * NP, NP)])
  plsc.subcore_barrier()

  stripe_cps = []
  for t in range(NS):
    stripe_cps.append(pltpu.async_copy(
        degp_hbm.at[pl.ds((cid * NS + t) * NP + sid * STR, STR)],
        sbuf.at[pl.ds(t * STR, STR)], sem_e))
  for cp in stripe_cps:
    cp.wait()

  def red(j, _):
    v = sbuf[pl.ds(j * L, L)]
    for t in range(1, NS):
      v = v + sbuf[pl.ds(t * STR + j * L, L)]
    dbuf[pl.ds(j * L, L)] = _rsqrt16(v + 1.0)
    return 0
  lax.fori_loop(0, STR // L, red, 0)

  pltpu.sync_copy(dbuf, dinvp_hbm.at[pl.ds(cid * NP + sid * STR, STR)])

  @pl.when(cid == 0)
  def _():
    def expand(j, _):
      d2buf[pl.ds(j * L, L)] = plsc.load_gather(dbuf, [j * 8 + half_io])
      return 0
    lax.fori_loop(0, 2 * STR // L, expand, 0)

    @pl.when(sid < NS - 1)
    def _():
      pltpu.sync_copy(d2buf, dinv2_hbm.at[pl.ds(sid * 2 * STR, 2 * STR)])

    @pl.when(sid == NS - 1)
    def _():
      pltpu.sync_copy(d2buf.at[pl.ds(0, 2 * N - (NS - 1) * 2 * STR)],
                      dinv2_hbm.at[pl.ds((NS - 1) * 2 * STR,
                                         2 * N - (NS - 1) * 2 * STR)])

  plsc.subcore_barrier()
  pltpu.sync_copy(dinvp_hbm.at[pl.ds(cid * NP, NP)], deg)

  cp_h.wait()
  balt = plsc.load_gather(b2, [io & 1])
  is_w0 = lax.broadcast(wid, (L,)) == 0

  def scale(j, _):
    sl = pl.ds(j * L, L)
    dv = plsc.load_gather(deg, [j * 8 + half_io])
    hv = hd[sl] * dv
    hd[sl] = hv
    acc[sl] = jnp.where(is_w0, hv + balt / dv, 0.0)
    return 0
  lax.fori_loop(0, 2 * N // L, scale, 0)

  cp_s.wait()
  cp_d2.wait()
  cp_w2.wait()

  def p2(i, _):
    s = srcv[pl.ds(i * L, L)]
    d = dstv[pl.ds(i * L, L)]
    w = ewv[pl.ds(i * L, L)]
    s2 = s << 1
    d2 = d << 1
    m0 = plsc.load_gather(hd, [s2]) * w
    m1 = plsc.load_gather(hd, [s2 + 1]) * w
    plsc.addupdate_scatter(acc, [d2], m0)
    plsc.addupdate_scatter(acc, [d2 + 1], m1)
    return 0
  lax.fori_loop(0, E2 // L, p2, 0)

  pltpu.sync_copy(acc, accp_hbm.at[wid])


@functools.partial(
    pl.kernel,
    out_type=(
        jax.ShapeDtypeStruct((NW, 2 * N), jnp.float32),
        jax.ShapeDtypeStruct((2 * N,), jnp.float32),
        jax.ShapeDtypeStruct((NW * NP,), jnp.float32),
        jax.ShapeDtypeStruct((NC * NP,), jnp.float32),
    ),
    mesh=_sc_mesh,
    scratch_types=[
        pltpu.VMEM((E1,), jnp.int32),
        pltpu.VMEM((E1,), jnp.float32),
        pltpu.VMEM((E2,), jnp.int32),
        pltpu.VMEM((NP,), jnp.float32),
        pltpu.VMEM((NP,), jnp.float32),
        pltpu.VMEM((STR,), jnp.float32),
        pltpu.VMEM((2 * STR,), jnp.float32),
        pltpu.VMEM((2 * N,), jnp.float32),
        pltpu.VMEM((2 * N,), jnp.float32),
        pltpu.VMEM((2,), jnp.float32),
        pltpu.SemaphoreType.DMA,
        pltpu.SemaphoreType.DMA,
        pltpu.SemaphoreType.DMA,
        pltpu.SemaphoreType.DMA,
        pltpu.SemaphoreType.DMA,
    ],
    compiler_params=pltpu.CompilerParams(needs_layout_passes=False),
)
def _sc_main(*refs):
  _sc_main_body(*refs)


def _tc_mm_body(x_ref, w1_ref, h_ref):
  h_ref[...] = jnp.dot(x_ref[...], w1_ref[...],
                       preferred_element_type=jnp.float32)


def _tc_mm(x, w1):
  return pl.pallas_call(
      _tc_mm_body,
      out_shape=jax.ShapeDtypeStruct((N, 2), jnp.float32),
  )(x, w1)


def _tc_final_body(accp_ref, dinv2_ref, wfc_ref, bfc_ref, o_ref):
  acc = jnp.sum(accp_ref[...], axis=0)
  out = jnp.maximum(dinv2_ref[...] * acc, 0.0)
  logit = jnp.sum(out * wfc_ref[...]) + bfc_ref[0]
  o_ref[0, 0] = 1.0 / (1.0 + jnp.exp(-logit))


def _tc_final(accp, dinv2, wfc, bfc):
  vm = pl.BlockSpec(memory_space=pltpu.VMEM)
  sm = pl.BlockSpec(memory_space=pltpu.SMEM)
  return pl.pallas_call(
      _tc_final_body,
      in_specs=[vm, vm, vm, sm],
      out_specs=sm,
      out_shape=jax.ShapeDtypeStruct((1, 1), jnp.float32),
  )(accp, dinv2, wfc, bfc)


def kernel(x, edge_list, edge_attr, W1, b1, Wfc, bfc):
  src = edge_list[0]
  dst = edge_list[1]

  h = _tc_mm(x, W1)
  hf = h.reshape(2 * N)
  accp, dinv2, _, _ = _sc_main(src, dst, edge_attr, hf, b1)

  out = _tc_final(
      accp.reshape(NW, 125, 160),
      dinv2.reshape(125, 160),
      Wfc.reshape(125, 160),
      bfc,
  )
  return out.reshape(())


# speedup vs baseline: 1.3509x; 1.3509x over previous
import functools

import jax
import jax.numpy as jnp
from jax import lax
from jax.experimental import pallas as pl
from jax.experimental.pallas import tpu as pltpu
from jax.experimental.pallas import tpu_sc as plsc

N = 10000
E = 320000
L = 16
NC = 2
NS = 16
NW = NC * NS
NP = 10240
STR = NP // NS
E1 = E // NS
E2 = E // NW

_sc_mesh = plsc.VectorSubcoreMesh(
    core_axis_name="c", subcore_axis_name="s", num_cores=NC, num_subcores=NS)


def _rsqrt16(v):
  i = plsc.bitcast(v, jnp.int32)
  y = plsc.bitcast(jnp.int32(0x5F3759DF) - (i >> 1), jnp.float32)
  for _ in range(3):
    y = y * (1.5 - 0.5 * v * y * y)
  return y


def _sc_main_body(src_hbm, dst_hbm, ew_hbm, hp_hbm,
                  accp0_hbm, accp1_hbm, dinv_hbm,
                  dstv, ewv, srcv, deg, sbuf, dbuf, h0d, h1d, acc0, acc1,
                  degparts, dinv_sh,
                  sem_a, sem_b, sem_c, sem_d, sem_e):
  cid = lax.axis_index("c")
  sid = lax.axis_index("s")
  wid = cid * NS + sid

  cp_d = pltpu.async_copy(dst_hbm.at[pl.ds(sid * E1, E1)], dstv, sem_a)
  cp_w = pltpu.async_copy(ew_hbm.at[pl.ds(sid * E1, E1)], ewv, sem_b)

  def zero_deg(i, _):
    deg[pl.ds(i * L, L)] = jnp.zeros((L,), jnp.float32)
    return 0
  lax.fori_loop(0, NP // L, zero_deg, 0)
  cp_d.wait()
  cp_w.wait()

  def p1(i, _):
    d = dstv[pl.ds(i * L, L)]
    w = ewv[pl.ds(i * L, L)]
    plsc.addupdate_scatter(deg, [d], w)
    return 0
  lax.fori_loop(0, E1 // L, p1, 0)

  cp_s = pltpu.async_copy(src_hbm.at[pl.ds(wid * E2, E2)], srcv, sem_a)
  cp_d2 = pltpu.async_copy(dst_hbm.at[pl.ds(wid * E2, E2)],
                           dstv.at[pl.ds(0, E2)], sem_b)
  cp_w2 = pltpu.async_copy(ew_hbm.at[pl.ds(wid * E2, E2)],
                           ewv.at[pl.ds(0, E2)], sem_c)
  cp_h0 = pltpu.async_copy(hp_hbm.at[0], h0d, sem_d)
  cp_h1 = pltpu.async_copy(hp_hbm.at[1], h1d, sem_e)

  pltpu.sync_copy(deg, degparts.at[pl.ds(sid * NP, NP)])
  plsc.subcore_barrier()

  for t in range(NS):
    pltpu.sync_copy(degparts.at[pl.ds(t * NP + sid * STR, STR)],
                    sbuf.at[pl.ds(t * STR, STR)])

  def red(j, _):
    v = sbuf[pl.ds(j * L, L)]
    for t in range(1, NS):
      v = v + sbuf[pl.ds(t * STR + j * L, L)]
    dbuf[pl.ds(j * L, L)] = _rsqrt16(v + 1.0)
    return 0
  lax.fori_loop(0, STR // L, red, 0)

  pltpu.sync_copy(dbuf, dinv_sh.at[pl.ds(sid * STR, STR)])
  plsc.subcore_barrier()

  pltpu.sync_copy(dinv_sh, deg)
  cp_h0.wait()
  cp_h1.wait()

  def scale(i, _):
    sl = pl.ds(i * L, L)
    dv = deg[sl]
    h0d[sl] = h0d[sl] * dv
    h1d[sl] = h1d[sl] * dv
    acc0[sl] = jnp.zeros((L,), jnp.float32)
    acc1[sl] = jnp.zeros((L,), jnp.float32)
    return 0
  lax.fori_loop(0, N // L, scale, 0)

  cp_s.wait()
  cp_d2.wait()
  cp_w2.wait()

  def p2(i, _):
    s = srcv[pl.ds(i * L, L)]
    d = dstv[pl.ds(i * L, L)]
    w = ewv[pl.ds(i * L, L)]
    m0 = plsc.load_gather(h0d, [s]) * w
    m1 = plsc.load_gather(h1d, [s]) * w
    plsc.addupdate_scatter(acc0, [d], m0)
    plsc.addupdate_scatter(acc1, [d], m1)
    return 0
  lax.fori_loop(0, E2 // L, p2, 0)

  pltpu.sync_copy(acc0, accp0_hbm.at[wid])
  pltpu.sync_copy(acc1, accp1_hbm.at[wid])

  @pl.when(wid == 0)
  def _():
    pltpu.sync_copy(deg.at[pl.ds(0, N)], dinv_hbm)


@functools.partial(
    pl.kernel,
    out_type=(
        jax.ShapeDtypeStruct((NW, N), jnp.float32),
        jax.ShapeDtypeStruct((NW, N), jnp.float32),
        jax.ShapeDtypeStruct((N,), jnp.float32),
    ),
    mesh=_sc_mesh,
    scratch_types=[
        pltpu.VMEM((E1,), jnp.int32),
        pltpu.VMEM((E1,), jnp.float32),
        pltpu.VMEM((E2,), jnp.int32),
        pltpu.VMEM((NP,), jnp.float32),
        pltpu.VMEM((NP,), jnp.float32),
        pltpu.VMEM((STR,), jnp.float32),
        pltpu.VMEM((N,), jnp.float32),
        pltpu.VMEM((N,), jnp.float32),
        pltpu.VMEM((N,), jnp.float32),
        pltpu.VMEM((N,), jnp.float32),
        pltpu.VMEM_SHARED((NS * NP,), jnp.float32),
        pltpu.VMEM_SHARED((NP,), jnp.float32),
        pltpu.SemaphoreType.DMA,
        pltpu.SemaphoreType.DMA,
        pltpu.SemaphoreType.DMA,
        pltpu.SemaphoreType.DMA,
        pltpu.SemaphoreType.DMA,
    ],
    compiler_params=pltpu.CompilerParams(needs_layout_passes=False),
)
def _sc_main(*refs):
  _sc_main_body(*refs)


def _tc_mm_body(w1t_ref, x_ref, hp_ref):
  hp_ref[...] = lax.dot_general(
      w1t_ref[...], x_ref[...], (((1,), (1,)), ((), ())),
      preferred_element_type=jnp.float32)


def _tc_mm(w1t, x):
  return pl.pallas_call(
      _tc_mm_body,
      out_shape=jax.ShapeDtypeStruct((2, N), jnp.float32),
  )(w1t, x)


def _tc_final_body(accp0_ref, accp1_ref, h0_ref, h1_ref, dinv_ref,
                   wfc0_ref, wfc1_ref, b1_ref, bfc_ref, o_ref):
  dv = dinv_ref[...]
  a0 = jnp.sum(accp0_ref[...], axis=0)
  a1 = jnp.sum(accp1_ref[...], axis=0)
  o0 = jnp.maximum(dv * (a0 + h0_ref[...] * dv) + b1_ref[0], 0.0)
  o1 = jnp.maximum(dv * (a1 + h1_ref[...] * dv) + b1_ref[1], 0.0)
  logit = (jnp.sum(o0 * wfc0_ref[...]) + jnp.sum(o1 * wfc1_ref[...])
           + bfc_ref[0])
  o_ref[0, 0] = 1.0 / (1.0 + jnp.exp(-logit))


def _tc_final(accp0, accp1, h0, h1, dinv, wfc0, wfc1, b1, bfc):
  vm = pl.BlockSpec(memory_space=pltpu.VMEM)
  sm = pl.BlockSpec(memory_space=pltpu.SMEM)
  return pl.pallas_call(
      _tc_final_body,
      in_specs=[vm, vm, vm, vm, vm, vm, vm, sm, sm],
      out_specs=sm,
      out_shape=jax.ShapeDtypeStruct((1, 1), jnp.float32),
  )(accp0, accp1, h0, h1, dinv, wfc0, wfc1, b1, bfc)


def kernel(x, edge_list, edge_attr, W1, b1, Wfc, bfc):
  src = edge_list[0]
  dst = edge_list[1]
  ew = edge_attr

  hp = _tc_mm(W1.T, x)
  accp0, accp1, dinv = _sc_main(src, dst, ew, hp)

  wfcp = Wfc.reshape(N, 2).T
  out = _tc_final(
      accp0.reshape(NW, 80, 125),
      accp1.reshape(NW, 80, 125),
      hp[0].reshape(80, 125),
      hp[1].reshape(80, 125),
      dinv.reshape(80, 125),
      wfcp[0].reshape(80, 125),
      wfcp[1].reshape(80, 125),
      b1,
      bfc,
  )
  return out.reshape(())
